# Initial kernel scaffold; baseline (speedup 1.0000x reference)
import jax
import jax.numpy as jnp
from jax.experimental import pallas as pl


def kernel(edge_pos, edge_neg, emb_user, emb_item, rel_emb):
    raise NotImplementedError("write your pallas kernel here")


# speedup vs baseline: 3.0237x; 3.0237x over previous
import jax
import jax.numpy as jnp
from jax import lax
from jax.experimental import pallas as pl
from jax.experimental.pallas import tpu as pltpu
from jax.experimental.pallas import tpu_sc as plsc

NC = 2
NS = 16
L = 16
W = NC * NS


def _scale_body(x_ref, r_ref, o_ref):
    o_ref[...] = x_ref[...] * r_ref[...]


def _prescale_user(emb_user, rel_emb):
    n, d = emb_user.shape
    rb = 1024
    return pl.pallas_call(
        _scale_body,
        out_shape=jax.ShapeDtypeStruct((n, d), jnp.float32),
        grid=(pl.cdiv(n, rb),),
        in_specs=[
            pl.BlockSpec((rb, d), lambda i: (i, 0)),
            pl.BlockSpec((1, d), lambda i: (0, 0)),
        ],
        out_specs=pl.BlockSpec((rb, d), lambda i: (i, 0)),
    )(emb_user, rel_emb)


def _sc_score(user_s, emb_item, pk, E, *, B=128):
    D = user_s.shape[1]
    per_w = E // W
    cpw = per_w // B
    n_side = E // B

    def body(user_hbm, item_hbm, pk_hbm, out_hbm,
             idx0, idx1, rA0, rB0, rA1, rB1, ov, semA0, semB0, semA1, semB1):
        wid = lax.axis_index("s") * NC + lax.axis_index("c")
        idxbufs = (idx0, idx1)
        rbufs = ((rA0, rB0), (rA1, rB1))
        sems = ((semA0, semB0), (semA1, semB1))

        for j in range(2):
            c0 = j * n_side + wid * cpw

            def fire(c, b):
                pltpu.sync_copy(pk_hbm.at[c0 + c], idxbufs[b])
                pltpu.async_copy(user_hbm.at[idxbufs[b].at[0]], rbufs[b][0], sems[b][0])
                pltpu.async_copy(item_hbm.at[idxbufs[b].at[1]], rbufs[b][1], sems[b][1])

            def waitg(b):
                pltpu.make_async_copy(user_hbm.at[idxbufs[b].at[0]], rbufs[b][0], sems[b][0]).wait()
                pltpu.make_async_copy(item_hbm.at[idxbufs[b].at[1]], rbufs[b][1], sems[b][1]).wait()

            def compute(c, b, j=j):
                rA, rB = rbufs[b]

                def group_body(g, _):
                    eidx = (g * L + lax.iota(jnp.int32, L)).astype(jnp.int32)
                    accs = [jnp.zeros((L,), jnp.float32) for _ in range(4)]
                    for d in range(D):
                        dvec = jnp.full((L,), d, jnp.int32)
                        a = plsc.load_gather(rA, [eidx, dvec])
                        bvec = plsc.load_gather(rB, [eidx, dvec])
                        accs[d % 4] = accs[d % 4] + a * bvec
                    ov[pl.ds(g * L, L)] = (accs[0] + accs[1]) + (accs[2] + accs[3])
                    return 0

                lax.fori_loop(0, B // L, group_body, 0)
                base = wid * per_w + c * B
                pltpu.sync_copy(ov, out_hbm.at[j, pl.ds(base, B)])

            for b in range(2):
                fire(b, b)

            def outer(i, _, fire=fire, waitg=waitg, compute=compute):
                for b in range(2):
                    c = i * 2 + b
                    waitg(b)
                    compute(c, b)
                    fire(c + 2, b)
                return 0

            lax.fori_loop(0, cpw // 2 - 1, outer, 0)
            for b in range(2):
                c = cpw - 2 + b
                waitg(b)
                compute(c, b)

    mesh = plsc.VectorSubcoreMesh(
        core_axis_name="c", subcore_axis_name="s", num_cores=NC, num_subcores=NS)
    return pl.kernel(
        body,
        out_type=jax.ShapeDtypeStruct((2, E), jnp.float32),
        mesh=mesh,
        compiler_params=pltpu.CompilerParams(
            needs_layout_passes=False, use_tc_tiling_on_sc=False),
        scratch_types=[
            pltpu.VMEM((2, B), jnp.int32),
            pltpu.VMEM((2, B), jnp.int32),
            pltpu.VMEM((B, D), jnp.float32),
            pltpu.VMEM((B, D), jnp.float32),
            pltpu.VMEM((B, D), jnp.float32),
            pltpu.VMEM((B, D), jnp.float32),
            pltpu.VMEM((B,), jnp.float32),
            pltpu.SemaphoreType.DMA,
            pltpu.SemaphoreType.DMA,
            pltpu.SemaphoreType.DMA,
            pltpu.SemaphoreType.DMA,
        ],
    )(user_s, emb_item, pk)


@jax.jit
def kernel(edge_pos, edge_neg, emb_user, emb_item, rel_emb):
    E = edge_pos.shape[1]
    B = 128
    user_s = _prescale_user(emb_user, rel_emb)
    pk = jnp.concatenate(
        [edge_pos.reshape(2, E // B, B).transpose(1, 0, 2),
         edge_neg.reshape(2, E // B, B).transpose(1, 0, 2)], axis=0)
    return _sc_score(user_s, emb_item, pk, E, B=B)
